# baseline (speedup 1.0000x reference)
import jax
import jax.numpy as jnp
from jax import lax
from jax.experimental import pallas as pl
from jax.experimental.pallas import tpu as pltpu

_VMEM_LIMIT = 40 * 1024 * 1024
_BF = jnp.bfloat16
_F32 = jnp.float32


def _nn_up(x, s):
    if s == 1:
        return x
    h, w, c = x.shape
    x = jnp.broadcast_to(x[:, :, None, :], (h, w, s, c)).reshape(h, w * s, c)
    x = jnp.broadcast_to(x[:, None, :, :], (h, s, w * s, c)).reshape(h * s, w * s, c)
    return x


def _lat_top_kernel(x_ref, w_ref, o_ref):
    x = x_ref[0].astype(_BF)
    o_ref[0] = lax.dot_general(
        x, w_ref[...], (((0,), (0,)), ((), ())),
        preferred_element_type=_F32).astype(o_ref.dtype)


def _lat_top(x_flat, wt, tt):
    N, Cin, HW = x_flat.shape
    Cout = wt.shape[1]
    return pl.pallas_call(
        _lat_top_kernel,
        out_shape=jax.ShapeDtypeStruct((N, HW, Cout), _BF),
        grid=(N, HW // tt),
        in_specs=[pl.BlockSpec((1, Cin, tt), lambda n, t: (n, 0, t)),
                  pl.BlockSpec((Cin, Cout), lambda n, t: (0, 0))],
        out_specs=pl.BlockSpec((1, tt, Cout), lambda n, t: (n, t, 0)),
        compiler_params=pltpu.CompilerParams(
            dimension_semantics=("parallel", "parallel"),
            vmem_limit_bytes=_VMEM_LIMIT),
        cost_estimate=pl.CostEstimate(
            flops=2 * N * HW * Cin * Cout, transcendentals=0,
            bytes_accessed=4 * N * Cin * HW + 2 * N * HW * Cout),
    )(x_flat, wt)


def _lat_up_add_kernel(x_ref, w_ref, c_ref, o_ref):
    thf, wf, cout = o_ref.shape[1], o_ref.shape[2], o_ref.shape[3]
    x = x_ref[0].astype(_BF)
    lat = lax.dot_general(
        x, w_ref[...], (((0,), (0,)), ((), ())),
        preferred_element_type=_F32).reshape(thf, wf, cout)
    up = _nn_up(c_ref[0].astype(_F32), 2)
    o_ref[0] = (lat + up).astype(o_ref.dtype)


def _lat_up_add(x_flat, wt, coarse, *, thc=8):
    N, Cin, HWf = x_flat.shape
    Nc, Hc, Wc, Cout = coarse.shape
    Hf, Wf = 2 * Hc, 2 * Wc
    thc = min(thc, Hc)
    thf = 2 * thc
    return pl.pallas_call(
        _lat_up_add_kernel,
        out_shape=jax.ShapeDtypeStruct((N, Hf, Wf, Cout), _BF),
        grid=(N, Hc // thc),
        in_specs=[pl.BlockSpec((1, Cin, thf * Wf), lambda n, h: (n, 0, h)),
                  pl.BlockSpec((Cin, Cout), lambda n, h: (0, 0)),
                  pl.BlockSpec((1, thc, Wc, Cout), lambda n, h: (n, h, 0, 0))],
        out_specs=pl.BlockSpec((1, thf, Wf, Cout), lambda n, h: (n, h, 0, 0)),
        compiler_params=pltpu.CompilerParams(
            dimension_semantics=("parallel", "parallel"),
            vmem_limit_bytes=_VMEM_LIMIT),
        cost_estimate=pl.CostEstimate(
            flops=2 * N * HWf * Cin * Cout, transcendentals=0,
            bytes_accessed=4 * N * Cin * HWf + N * HWf * Cout * 2 * 2),
    )(x_flat, wt, coarse)


_SCALES = (8, 4, 2, 1)
_TH2 = 8


def _fuse_kernel(c5b, c4b, c3b, c2b, x5, x4, x3, x2, w_ref, o_ref,
                 h5, h4, h3, h2, s5, s4, s3, s2, sems):
    n = pl.program_id(0)
    hb = pl.program_id(1)
    nblk = pl.num_programs(1)
    ctr = (c5b, c4b, c3b, c2b)
    halos = (h5, h4, h3, h2)
    stks = (s5, s4, s3, s2)
    srcs = (x5, x4, x3, x2)
    copies = []
    for idx in range(4):
        th = _TH2 // _SCALES[idx]
        xr, hr = srcs[idx], halos[idx]
        H = xr.shape[1]
        r0 = hb * th
        ct = pltpu.make_async_copy(xr.at[n, pl.ds(jnp.maximum(r0 - 1, 0), 1)],
                                   hr.at[pl.ds(0, 1)], sems.at[2 * idx])
        cb = pltpu.make_async_copy(xr.at[n, pl.ds(jnp.minimum(r0 + th, H - 1), 1)],
                                   hr.at[pl.ds(1, 1)], sems.at[2 * idx + 1])
        ct.start()
        cb.start()
        copies.append((ct, cb))
    for idx in range(4):
        scale = _SCALES[idx]
        th = _TH2 // scale
        xc, hr, Br = ctr[idx], halos[idx], stks[idx]
        W = srcs[idx].shape[2]
        T = th * W
        Br[:, 256:512] = xc[0].reshape(T, 256)
        if th > 1:
            Br[W:, 0:256] = xc[0, 0:th - 1].reshape(T - W, 256)
            Br[:T - W, 512:768] = xc[0, 1:th].reshape(T - W, 256)
        ct, cb = copies[idx]
        ct.wait()
        cb.wait()

        @pl.when(hb == 0)
        def _top(hr=hr, W=W):
            hr[0:1] = jnp.zeros((1, W, 256), hr.dtype)

        @pl.when(hb == nblk - 1)
        def _bot(hr=hr, W=W):
            hr[1:2] = jnp.zeros((1, W, 256), hr.dtype)

        Br[0:W, 0:256] = hr[0:1].reshape(W, 256)
        Br[T - W:, 512:768] = hr[1:2].reshape(W, 256)
        S = jnp.dot(Br[...], w_ref[idx],
                    preferred_element_type=_F32).reshape(th, W, 384)
        z = jnp.zeros((th, 1, 128), _F32)
        y = (S[:, :, 128:256]
             + jnp.concatenate([z, S[:, :-1, 0:128]], axis=1)
             + jnp.concatenate([S[:, 1:, 256:384], z], axis=1))
        up = _nn_up(y, scale)
        hw = up.shape[0] * up.shape[1]
        yt = jnp.transpose(up.reshape(hw, 128))[0:64]
        o_ref[0, 64 * idx:64 * (idx + 1), :] = yt.astype(o_ref.dtype)


def _fused_convs_concat(in5, out4, out3, out2, wstk):
    N, H2, W2, Cb = out2.shape[0], out2.shape[1], out2.shape[2], 64
    flops = sum(2 * 9 * N * (H2 // s) * (W2 // s) * 256 * 64 for s in _SCALES)
    halos = [pltpu.VMEM((2, W2 // s, 256), _BF) for s in _SCALES]
    stks = [pltpu.VMEM(((_TH2 // s) * (W2 // s), 768), _BF) for s in _SCALES]
    ctr_specs = [
        pl.BlockSpec((1, _TH2 // s, W2 // s, 256), lambda n, h: (n, h, 0, 0))
        for s in _SCALES]
    return pl.pallas_call(
        _fuse_kernel,
        out_shape=jax.ShapeDtypeStruct((N, 4 * Cb, H2 * W2), _F32),
        grid=(N, H2 // _TH2),
        in_specs=ctr_specs + [
            pl.BlockSpec(memory_space=pl.ANY),
            pl.BlockSpec(memory_space=pl.ANY),
            pl.BlockSpec(memory_space=pl.ANY),
            pl.BlockSpec(memory_space=pl.ANY),
            pl.BlockSpec((4, 768, 384), lambda n, h: (0, 0, 0))],
        out_specs=pl.BlockSpec((1, 4 * Cb, _TH2 * W2), lambda n, h: (n, 0, h)),
        scratch_shapes=halos + stks + [pltpu.SemaphoreType.DMA((8,))],
        compiler_params=pltpu.CompilerParams(
            dimension_semantics=("parallel", "parallel"),
            vmem_limit_bytes=_VMEM_LIMIT),
        cost_estimate=pl.CostEstimate(
            flops=flops, transcendentals=0,
            bytes_accessed=4 * N * H2 * W2 * 4 * Cb
            + 2 * N * (H2 * W2 + 3 * (H2 // 2) * (W2 // 2)) * 256),
    )(in5, out4, out3, out2, in5, out4, out3, out2, wstk)


def _mk_conv_w(p):
    wt = jnp.transpose(p, (2, 3, 1, 0))
    wt = jnp.pad(wt, ((0, 0), (0, 0), (0, 0), (0, 64)))
    return jnp.transpose(wt, (0, 2, 1, 3)).reshape(768, 384).astype(_BF)


def kernel(c2, c3, c4, c5, in2, in3, in4, in5, p5, p4, p3, p2):
    N = c2.shape[0]
    w5t = jnp.transpose(in5).astype(_BF)
    w4t = jnp.transpose(in4).astype(_BF)
    w3t = jnp.transpose(in3).astype(_BF)
    w2t = jnp.transpose(in2).astype(_BF)
    wstk = jnp.stack([_mk_conv_w(p5), _mk_conv_w(p4),
                      _mk_conv_w(p3), _mk_conv_w(p2)])

    c5f = c5.reshape(N, c5.shape[1], -1)
    c4f = c4.reshape(N, c4.shape[1], -1)
    c3f = c3.reshape(N, c3.shape[1], -1)
    c2f = c2.reshape(N, c2.shape[1], -1)

    t5 = _lat_top(c5f, w5t, min(512, c5f.shape[2]))
    t5 = t5.reshape(N, c5.shape[2], c5.shape[3], 256)
    o4 = _lat_up_add(c4f, w4t, t5)
    o3 = _lat_up_add(c3f, w3t, o4)
    o2 = _lat_up_add(c2f, w2t, o3)

    fuse = _fused_convs_concat(t5, o4, o3, o2, wstk)
    return fuse.reshape(N, 256, c2.shape[2], c2.shape[3])


# speedup vs baseline: 1.2285x; 1.2285x over previous
import jax
import jax.numpy as jnp
from jax import lax
from jax.experimental import pallas as pl
from jax.experimental.pallas import tpu as pltpu

_VMEM_LIMIT = 56 * 1024 * 1024
_BF = jnp.bfloat16
_F32 = jnp.float32


def _nn_up(x, s):
    if s == 1:
        return x
    h, w, c = x.shape
    x = jnp.broadcast_to(x[:, :, None, :], (h, w, s, c)).reshape(h, w * s, c)
    x = jnp.broadcast_to(x[:, None, :, :], (h, s, w * s, c)).reshape(h * s, w * s, c)
    return x


def _lat_top_kernel(x_ref, w_ref, o_ref):
    x = x_ref[0].astype(_BF)
    o_ref[0] = lax.dot_general(
        x, w_ref[...], (((0,), (0,)), ((), ())),
        preferred_element_type=_F32).astype(o_ref.dtype)


def _lat_top(x_flat, wt, tt):
    N, Cin, HW = x_flat.shape
    Cout = wt.shape[1]
    return pl.pallas_call(
        _lat_top_kernel,
        out_shape=jax.ShapeDtypeStruct((N, HW, Cout), _BF),
        grid=(N, HW // tt),
        in_specs=[pl.BlockSpec((1, Cin, tt), lambda n, t: (n, 0, t)),
                  pl.BlockSpec((Cin, Cout), lambda n, t: (0, 0))],
        out_specs=pl.BlockSpec((1, tt, Cout), lambda n, t: (n, t, 0)),
        compiler_params=pltpu.CompilerParams(
            dimension_semantics=("parallel", "parallel"),
            vmem_limit_bytes=_VMEM_LIMIT),
        cost_estimate=pl.CostEstimate(
            flops=2 * N * HW * Cin * Cout, transcendentals=0,
            bytes_accessed=4 * N * Cin * HW + 2 * N * HW * Cout),
    )(x_flat, wt)


def _lat_up_add_kernel(x_ref, w_ref, c_ref, o_ref):
    thf, wf, cout = o_ref.shape[1], o_ref.shape[2], o_ref.shape[3]
    x = x_ref[0].astype(_BF)
    lat = lax.dot_general(
        x, w_ref[...], (((0,), (0,)), ((), ())),
        preferred_element_type=_F32).reshape(thf, wf, cout)
    up = _nn_up(c_ref[0].astype(_F32), 2)
    o_ref[0] = (lat + up).astype(o_ref.dtype)


def _lat_up_add(x_flat, wt, coarse, *, thc=8):
    N, Cin, HWf = x_flat.shape
    Nc, Hc, Wc, Cout = coarse.shape
    Hf, Wf = 2 * Hc, 2 * Wc
    thc = min(thc, Hc)
    thf = 2 * thc
    return pl.pallas_call(
        _lat_up_add_kernel,
        out_shape=jax.ShapeDtypeStruct((N, Hf, Wf, Cout), _BF),
        grid=(N, Hc // thc),
        in_specs=[pl.BlockSpec((1, Cin, thf * Wf), lambda n, h: (n, 0, h)),
                  pl.BlockSpec((Cin, Cout), lambda n, h: (0, 0)),
                  pl.BlockSpec((1, thc, Wc, Cout), lambda n, h: (n, h, 0, 0))],
        out_specs=pl.BlockSpec((1, thf, Wf, Cout), lambda n, h: (n, h, 0, 0)),
        compiler_params=pltpu.CompilerParams(
            dimension_semantics=("parallel", "parallel"),
            vmem_limit_bytes=_VMEM_LIMIT),
        cost_estimate=pl.CostEstimate(
            flops=2 * N * HWf * Cin * Cout, transcendentals=0,
            bytes_accessed=4 * N * Cin * HWf + N * HWf * Cout * 2 * 2),
    )(x_flat, wt, coarse)


_SCALES = (8, 4, 2, 1)
_TH2 = 16


def _fuse_kernel(c5p, c5c, c5n, c4p, c4c, c4n, c3p, c3c, c3n, c2p, c2c, c2n,
                 w_ref, o_ref, s5, s4, s3, s2):
    hb = pl.program_id(1)
    nblk = pl.num_programs(1)
    trips = ((c5p, c5c, c5n), (c4p, c4c, c4n), (c3p, c3c, c3n), (c2p, c2c, c2n))
    stks = (s5, s4, s3, s2)
    for idx in range(4):
        scale = _SCALES[idx]
        th = _TH2 // scale
        xp, xc, xn = trips[idx]
        Br = stks[idx]
        W = xc.shape[2]
        T = th * W
        Br[:, 256:512] = xc[0].reshape(T, 256)
        Br[W:, 0:256] = xc[0, 0:th - 1].reshape(T - W, 256)
        Br[:T - W, 512:768] = xc[0, 1:th].reshape(T - W, 256)
        Br[0:W, 0:256] = jnp.where(hb == 0, jnp.zeros((), Br.dtype),
                                   xp[0, th - 1].reshape(W, 256))
        Br[T - W:, 512:768] = jnp.where(hb == nblk - 1, jnp.zeros((), Br.dtype),
                                        xn[0, 0].reshape(W, 256))
        S = jnp.dot(Br[...], w_ref[idx],
                    preferred_element_type=_F32).reshape(th, W, 384)
        z = jnp.zeros((th, 1, 128), _F32)
        y = (S[:, :, 128:256]
             + jnp.concatenate([z, S[:, :-1, 0:128]], axis=1)
             + jnp.concatenate([S[:, 1:, 256:384], z], axis=1))
        up = _nn_up(y, scale)
        hw = up.shape[0] * up.shape[1]
        yt = jnp.transpose(up.reshape(hw, 128))[0:64]
        o_ref[0, 64 * idx:64 * (idx + 1), :] = yt.astype(o_ref.dtype)


def _fused_convs_concat(in5, out4, out3, out2, wstk):
    N, H2, W2, Cb = out2.shape[0], out2.shape[1], out2.shape[2], 64
    flops = sum(2 * 9 * N * (H2 // s) * (W2 // s) * 256 * 64 for s in _SCALES)
    stks = [pltpu.VMEM(((_TH2 // s) * (W2 // s), 768), _BF) for s in _SCALES]
    specs = []
    for s in _SCALES:
        bs = (1, _TH2 // s, W2 // s, 256)
        specs += [
            pl.BlockSpec(bs, lambda n, h: (n, jnp.maximum(h - 1, 0), 0, 0)),
            pl.BlockSpec(bs, lambda n, h: (n, h, 0, 0)),
            pl.BlockSpec(bs, lambda n, h, nb=H2 // _TH2:
                         (n, jnp.minimum(h + 1, nb - 1), 0, 0)),
        ]
    return pl.pallas_call(
        _fuse_kernel,
        out_shape=jax.ShapeDtypeStruct((N, 4 * Cb, H2 * W2), _F32),
        grid=(N, H2 // _TH2),
        in_specs=specs + [pl.BlockSpec((4, 768, 384), lambda n, h: (0, 0, 0))],
        out_specs=pl.BlockSpec((1, 4 * Cb, _TH2 * W2), lambda n, h: (n, 0, h)),
        scratch_shapes=stks,
        compiler_params=pltpu.CompilerParams(
            dimension_semantics=("parallel", "parallel"),
            vmem_limit_bytes=_VMEM_LIMIT),
        cost_estimate=pl.CostEstimate(
            flops=flops, transcendentals=0,
            bytes_accessed=4 * N * H2 * W2 * 4 * Cb
            + 3 * 2 * N * (H2 * W2 + 3 * (H2 // 2) * (W2 // 2)) * 256),
    )(*([in5, out4, out3, out2][i // 3] for i in range(12)), wstk)


def _mk_conv_w(p):
    wt = jnp.transpose(p, (2, 3, 1, 0))
    wt = jnp.pad(wt, ((0, 0), (0, 0), (0, 0), (0, 64)))
    return jnp.transpose(wt, (0, 2, 1, 3)).reshape(768, 384).astype(_BF)


def kernel(c2, c3, c4, c5, in2, in3, in4, in5, p5, p4, p3, p2):
    N = c2.shape[0]
    w5t = jnp.transpose(in5).astype(_BF)
    w4t = jnp.transpose(in4).astype(_BF)
    w3t = jnp.transpose(in3).astype(_BF)
    w2t = jnp.transpose(in2).astype(_BF)
    wstk = jnp.stack([_mk_conv_w(p5), _mk_conv_w(p4),
                      _mk_conv_w(p3), _mk_conv_w(p2)])

    c5f = c5.reshape(N, c5.shape[1], -1)
    c4f = c4.reshape(N, c4.shape[1], -1)
    c3f = c3.reshape(N, c3.shape[1], -1)
    c2f = c2.reshape(N, c2.shape[1], -1)

    t5 = _lat_top(c5f, w5t, min(512, c5f.shape[2]))
    t5 = t5.reshape(N, c5.shape[2], c5.shape[3], 256)
    o4 = _lat_up_add(c4f, w4t, t5)
    o3 = _lat_up_add(c3f, w3t, o4)
    o2 = _lat_up_add(c2f, w2t, o3)

    fuse = _fused_convs_concat(t5, o4, o3, o2, wstk)
    return fuse.reshape(N, 256, c2.shape[2], c2.shape[3])


# speedup vs baseline: 1.2920x; 1.0517x over previous
import jax
import jax.numpy as jnp
from jax import lax
from jax.experimental import pallas as pl
from jax.experimental.pallas import tpu as pltpu

_VMEM_LIMIT = 56 * 1024 * 1024
_BF = jnp.bfloat16
_F32 = jnp.float32


def _nn_up(x, s):
    if s == 1:
        return x
    h, w, c = x.shape
    x = jnp.broadcast_to(x[:, :, None, :], (h, w, s, c)).reshape(h, w * s, c)
    x = jnp.broadcast_to(x[:, None, :, :], (h, s, w * s, c)).reshape(h * s, w * s, c)
    return x


def _lat_top_kernel(x_ref, w_ref, o_ref):
    x = x_ref[0].astype(_BF)
    o_ref[0] = lax.dot_general(
        x, w_ref[...], (((0,), (0,)), ((), ())),
        preferred_element_type=_F32).astype(o_ref.dtype)


def _lat_top(x_flat, wt, tt):
    N, Cin, HW = x_flat.shape
    Cout = wt.shape[1]
    return pl.pallas_call(
        _lat_top_kernel,
        out_shape=jax.ShapeDtypeStruct((N, HW, Cout), _BF),
        grid=(N, HW // tt),
        in_specs=[pl.BlockSpec((1, Cin, tt), lambda n, t: (n, 0, t)),
                  pl.BlockSpec((Cin, Cout), lambda n, t: (0, 0))],
        out_specs=pl.BlockSpec((1, tt, Cout), lambda n, t: (n, t, 0)),
        compiler_params=pltpu.CompilerParams(
            dimension_semantics=("parallel", "parallel"),
            vmem_limit_bytes=_VMEM_LIMIT),
        cost_estimate=pl.CostEstimate(
            flops=2 * N * HW * Cin * Cout, transcendentals=0,
            bytes_accessed=4 * N * Cin * HW + 2 * N * HW * Cout),
    )(x_flat, wt)


def _lat_chain_kernel(c4_ref, c3_ref, c2_ref, t5_ref, w4_ref, w3_ref, w2_ref,
                      o4_ref, o3_ref, o2_ref):
    def lat(c_ref, w_ref, o_shape):
        x = c_ref[0].astype(_BF)
        return lax.dot_general(
            x, w_ref[...], (((0,), (0,)), ((), ())),
            preferred_element_type=_F32).reshape(o_shape)
    v4 = lat(c4_ref, w4_ref, o4_ref.shape[1:]) + _nn_up(
        t5_ref[0].astype(_F32), 2)
    o4_ref[0] = v4.astype(o4_ref.dtype)
    v3 = lat(c3_ref, w3_ref, o3_ref.shape[1:]) + _nn_up(v4, 2)
    o3_ref[0] = v3.astype(o3_ref.dtype)
    v2 = lat(c2_ref, w2_ref, o2_ref.shape[1:]) + _nn_up(v3, 2)
    o2_ref[0] = v2.astype(o2_ref.dtype)


def _lat_chain(c4f, c3f, c2f, t5, w4t, w3t, w2t, *, th2=16):
    N = c2f.shape[0]
    W2 = t5.shape[2] * 8
    H2 = c2f.shape[2] // W2
    H4, W4, H3, W3 = H2 // 4, W2 // 4, H2 // 2, W2 // 2
    th4, th3 = th2 // 4, th2 // 2
    grid = (N, H2 // th2)
    flops = 2 * N * 256 * (H2 * W2 * 256 + H3 * W3 * 512 + H4 * W4 * 1024)
    out_shapes = [jax.ShapeDtypeStruct((N, H4, W4, 256), _BF),
                  jax.ShapeDtypeStruct((N, H3, W3, 256), _BF),
                  jax.ShapeDtypeStruct((N, H2, W2, 256), _BF)]
    return pl.pallas_call(
        _lat_chain_kernel,
        out_shape=out_shapes,
        grid=grid,
        in_specs=[
            pl.BlockSpec((1, 1024, th4 * W4), lambda n, h: (n, 0, h)),
            pl.BlockSpec((1, 512, th3 * W3), lambda n, h: (n, 0, h)),
            pl.BlockSpec((1, 256, th2 * W2), lambda n, h: (n, 0, h)),
            pl.BlockSpec((1, th4 // 2, W4 // 2, 256), lambda n, h: (n, h, 0, 0)),
            pl.BlockSpec((1024, 256), lambda n, h: (0, 0)),
            pl.BlockSpec((512, 256), lambda n, h: (0, 0)),
            pl.BlockSpec((256, 256), lambda n, h: (0, 0)),
        ],
        out_specs=[
            pl.BlockSpec((1, th4, W4, 256), lambda n, h: (n, h, 0, 0)),
            pl.BlockSpec((1, th3, W3, 256), lambda n, h: (n, h, 0, 0)),
            pl.BlockSpec((1, th2, W2, 256), lambda n, h: (n, h, 0, 0)),
        ],
        compiler_params=pltpu.CompilerParams(
            dimension_semantics=("parallel", "parallel"),
            vmem_limit_bytes=_VMEM_LIMIT),
        cost_estimate=pl.CostEstimate(
            flops=flops, transcendentals=0,
            bytes_accessed=4 * (N * 1024 * H4 * W4 + N * 512 * H3 * W3
                                + N * 256 * H2 * W2)
            + 2 * N * 256 * (H4 * W4 + H3 * W3 + H2 * W2)),
    )(c4f, c3f, c2f, t5, w4t, w3t, w2t)


_SCALES = (8, 4, 2, 1)
_TH2 = 16


def _fuse_kernel(c5p, c5c, c5n, c4p, c4c, c4n, c3p, c3c, c3n, c2p, c2c, c2n,
                 w_ref, o_ref, s5, s4, s3, s2):
    hb = pl.program_id(1)
    nblk = pl.num_programs(1)
    trips = ((c5p, c5c, c5n), (c4p, c4c, c4n), (c3p, c3c, c3n), (c2p, c2c, c2n))
    stks = (s5, s4, s3, s2)
    for idx in range(4):
        scale = _SCALES[idx]
        th = _TH2 // scale
        xp, xc, xn = trips[idx]
        Br = stks[idx]
        W = xc.shape[2]
        T = th * W
        Br[:, 256:512] = xc[0].reshape(T, 256)
        Br[W:, 0:256] = xc[0, 0:th - 1].reshape(T - W, 256)
        Br[:T - W, 512:768] = xc[0, 1:th].reshape(T - W, 256)
        Br[0:W, 0:256] = jnp.where(hb == 0, jnp.zeros((), Br.dtype),
                                   xp[0, th - 1].reshape(W, 256))
        Br[T - W:, 512:768] = jnp.where(hb == nblk - 1, jnp.zeros((), Br.dtype),
                                        xn[0, 0].reshape(W, 256))
        S = jnp.dot(Br[...], w_ref[idx],
                    preferred_element_type=_F32).reshape(th, W, 384)
        z = jnp.zeros((th, 1, 128), _F32)
        y = (S[:, :, 128:256]
             + jnp.concatenate([z, S[:, :-1, 0:128]], axis=1)
             + jnp.concatenate([S[:, 1:, 256:384], z], axis=1))
        up = _nn_up(y, scale)
        hw = up.shape[0] * up.shape[1]
        yt = jnp.transpose(up.reshape(hw, 128))[0:64]
        o_ref[0, 64 * idx:64 * (idx + 1), :] = yt.astype(o_ref.dtype)


def _fused_convs_concat(in5, out4, out3, out2, wstk):
    N, H2, W2, Cb = out2.shape[0], out2.shape[1], out2.shape[2], 64
    flops = sum(2 * 9 * N * (H2 // s) * (W2 // s) * 256 * 64 for s in _SCALES)
    stks = [pltpu.VMEM(((_TH2 // s) * (W2 // s), 768), _BF) for s in _SCALES]
    specs = []
    for s in _SCALES:
        bs = (1, _TH2 // s, W2 // s, 256)
        specs += [
            pl.BlockSpec(bs, lambda n, h: (n, jnp.maximum(h - 1, 0), 0, 0)),
            pl.BlockSpec(bs, lambda n, h: (n, h, 0, 0)),
            pl.BlockSpec(bs, lambda n, h, nb=H2 // _TH2:
                         (n, jnp.minimum(h + 1, nb - 1), 0, 0)),
        ]
    return pl.pallas_call(
        _fuse_kernel,
        out_shape=jax.ShapeDtypeStruct((N, 4 * Cb, H2 * W2), _F32),
        grid=(N, H2 // _TH2),
        in_specs=specs + [pl.BlockSpec((4, 768, 384), lambda n, h: (0, 0, 0))],
        out_specs=pl.BlockSpec((1, 4 * Cb, _TH2 * W2), lambda n, h: (n, 0, h)),
        scratch_shapes=stks,
        compiler_params=pltpu.CompilerParams(
            dimension_semantics=("parallel", "parallel"),
            vmem_limit_bytes=_VMEM_LIMIT),
        cost_estimate=pl.CostEstimate(
            flops=flops, transcendentals=0,
            bytes_accessed=4 * N * H2 * W2 * 4 * Cb
            + 3 * 2 * N * (H2 * W2 + 3 * (H2 // 2) * (W2 // 2)) * 256),
    )(*([in5, out4, out3, out2][i // 3] for i in range(12)), wstk)


def _mk_conv_w(p):
    wt = jnp.transpose(p, (2, 3, 1, 0))
    wt = jnp.pad(wt, ((0, 0), (0, 0), (0, 0), (0, 64)))
    return jnp.transpose(wt, (0, 2, 1, 3)).reshape(768, 384).astype(_BF)


def kernel(c2, c3, c4, c5, in2, in3, in4, in5, p5, p4, p3, p2):
    N = c2.shape[0]
    w5t = jnp.transpose(in5).astype(_BF)
    w4t = jnp.transpose(in4).astype(_BF)
    w3t = jnp.transpose(in3).astype(_BF)
    w2t = jnp.transpose(in2).astype(_BF)
    wstk = jnp.stack([_mk_conv_w(p5), _mk_conv_w(p4),
                      _mk_conv_w(p3), _mk_conv_w(p2)])

    c5f = c5.reshape(N, c5.shape[1], -1)
    c4f = c4.reshape(N, c4.shape[1], -1)
    c3f = c3.reshape(N, c3.shape[1], -1)
    c2f = c2.reshape(N, c2.shape[1], -1)

    t5 = _lat_top(c5f, w5t, min(512, c5f.shape[2]))
    t5 = t5.reshape(N, c5.shape[2], c5.shape[3], 256)
    o4, o3, o2 = _lat_chain(c4f, c3f, c2f, t5, w4t, w3t, w2t,
                            th2=min(16, c2.shape[2]))

    fuse = _fused_convs_concat(t5, o4, o3, o2, wstk)
    return fuse.reshape(N, 256, c2.shape[2], c2.shape[3])
